# baseline (speedup 1.0000x reference)
import jax
import jax.numpy as jnp
from jax.experimental import pallas as pl

N = 10000
E = 320000
HID = 64
HEADS = 4


def _add_self_loops(edge_index, n):
    loop = jnp.arange(n, dtype=edge_index.dtype)
    src = jnp.concatenate([edge_index[0], loop])
    dst = jnp.concatenate([edge_index[1], loop])
    return src, dst


def _gcn_layer(x, edge_index, W, b, dinv):
    src, dst = _add_self_loops(edge_index, x.shape[0])
    norm = dinv[src] * dinv[dst]
    xw = x @ W
    msg = xw[src] * norm[:, None]
    out = jax.ops.segment_sum(msg, dst, num_segments=x.shape[0])
    return out + b


def _gat_layer(x, edge_index, Wg, att_src, att_dst, bg):
    n = x.shape[0]
    src, dst = _add_self_loops(edge_index, n)
    xg = (x @ Wg).reshape(n, HEADS, HID)
    a_src = (xg * att_src[None, :, :]).sum(-1)
    a_dst = (xg * att_dst[None, :, :]).sum(-1)
    e = jax.nn.leaky_relu(a_src[src] + a_dst[dst], 0.2)
    m = jax.ops.segment_max(e, dst, num_segments=n)
    ex = jnp.exp(e - m[dst])
    denom = jax.ops.segment_sum(ex, dst, num_segments=n)
    alpha = ex / (denom[dst] + 1e-16)
    out = jax.ops.segment_sum(xg[src] * alpha[:, :, None], dst, num_segments=n)
    return out.mean(axis=1) + bg


def _batch_norm(x, gamma, beta):
    mu = x.mean(axis=0)
    var = x.var(axis=0)
    return gamma * (x - mu) / jnp.sqrt(var + 1e-5) + beta


def _mlp_body(pooled_ref, fc1w_ref, fc1b_ref, fc2w_ref, fc2b_ref, out_ref):
    f = jnp.maximum(pooled_ref[...] @ fc1w_ref[...] + fc1b_ref[...], 0.0)
    out_ref[...] = f @ fc2w_ref[...] + fc2b_ref[...]


def kernel(x, edge_index, W1, b1, W2, b2, W3, b3, Wg, att_src, att_dst, bg, gamma, beta, fc1_w, fc1_b, fc2_w, fc2_b):
    n = x.shape[0]
    src, dst = _add_self_loops(edge_index, n)
    deg = jax.ops.segment_sum(jnp.ones(src.shape[0], dtype=x.dtype), dst, num_segments=n)
    dinv = jnp.where(deg > 0, deg ** -0.5, 0.0)

    h = jax.nn.relu(_gcn_layer(x, edge_index, W1, b1, dinv))
    h = _batch_norm(h, gamma, beta)
    h = jax.nn.relu(_gcn_layer(h, edge_index, W2, b2, dinv))
    h = _batch_norm(h, gamma, beta)
    h = jax.nn.relu(_gcn_layer(h, edge_index, W3, b3, dinv))
    h = jax.nn.relu(_gat_layer(h, edge_index, Wg, att_src, att_dst, bg))
    pooled = jnp.mean(h, axis=0, keepdims=True)

    out = pl.pallas_call(
        _mlp_body,
        out_shape=jax.ShapeDtypeStruct((1, fc2_w.shape[1]), jnp.float32),
    )(pooled, fc1_w, fc1_b, fc2_w, fc2_b)
    return out


# speedup vs baseline: 8.5985x; 8.5985x over previous
import functools

import jax
import jax.numpy as jnp
from jax import lax
from jax.experimental import pallas as pl
from jax.experimental.pallas import tpu as pltpu
from jax.experimental.pallas import tpu_sc as plsc

N = 10000
E = 320000
HID = 64
HEADS = 4

NC = 2
NS = 16
NW = NC * NS
NP = 10240
CH = 400
NCHUNK = E // CH
ES8 = 8
EPS8 = E // ES8
CPS8 = NCHUNK // ES8


def _mesh():
    return plsc.VectorSubcoreMesh(core_axis_name="c", subcore_axis_name="s")


_SC_PARAMS = pltpu.CompilerParams(needs_layout_passes=False)


def _zero_ref(ref, nrow, ncol):
    z = jnp.zeros((16,), jnp.float32)

    def zcol(col, carry):
        def zrow(i, carry2):
            ref[col, pl.ds(i * 16, 16)] = z
            return carry2
        lax.fori_loop(0, ncol // 16, zrow, 0)
        return carry
    lax.fori_loop(0, nrow, zcol, 0)


@functools.partial(
    pl.kernel,
    out_type=jax.ShapeDtypeStruct((NW, NP), jnp.float32),
    mesh=_mesh(),
    compiler_params=_SC_PARAMS,
    scratch_types=[
        pltpu.VMEM((CH,), jnp.int32),
        pltpu.VMEM((1, NP), jnp.float32),
    ],
)
def _deg_kernel(dst_hbm, out_hbm, didx, acc):
    c = lax.axis_index("c")
    s = lax.axis_index("s")
    wid = s * NC + c
    _zero_ref(acc, 1, NP)
    ones16 = jnp.ones((16,), jnp.float32)
    zcol = jnp.zeros((16,), jnp.int32)
    nch = NCHUNK // NW

    def body(i, carry):
        pltpu.sync_copy(dst_hbm.at[pl.ds((wid * nch + i) * CH, CH)], didx)
        for g in range(CH // 16):
            d16 = didx[pl.ds(g * 16, 16)]
            plsc.addupdate_scatter(acc, [zcol, d16], ones16)
        return carry

    lax.fori_loop(0, nch, body, 0)
    pltpu.sync_copy(acc.at[0], out_hbm.at[wid])


@functools.partial(
    pl.kernel,
    out_type=jax.ShapeDtypeStruct((NW, 4, NP), jnp.float32),
    mesh=_mesh(),
    compiler_params=_SC_PARAMS,
    scratch_types=[
        pltpu.VMEM((CH,), jnp.int32),
        pltpu.VMEM((CH,), jnp.int32),
        pltpu.VMEM((4, NP), jnp.float32),
        pltpu.VMEM((4, NP), jnp.float32),
    ],
)
def _gcn_scatter(src_hbm, dst_hbm, yt_hbm, out_hbm, sidx, didx, ytab, acc):
    c = lax.axis_index("c")
    s = lax.axis_index("s")
    wid = s * NC + c
    es2 = wid // 16
    cs = wid % 16
    pltpu.sync_copy(yt_hbm.at[cs], ytab)
    _zero_ref(acc, 4, NP)
    cols = [jnp.full((16,), col, jnp.int32) for col in range(4)]
    nch = NCHUNK // 2

    def body(i, carry):
        base = (es2 * nch + i) * CH
        pltpu.sync_copy(src_hbm.at[pl.ds(base, CH)], sidx)
        pltpu.sync_copy(dst_hbm.at[pl.ds(base, CH)], didx)
        for g in range(CH // 16):
            s16 = sidx[pl.ds(g * 16, 16)]
            d16 = didx[pl.ds(g * 16, 16)]
            for col in range(4):
                v = plsc.load_gather(ytab, [cols[col], s16])
                plsc.addupdate_scatter(acc, [cols[col], d16], v)
        return carry

    lax.fori_loop(0, nch, body, 0)
    pltpu.sync_copy(acc, out_hbm.at[wid])


def _gcn_layer_sc(h, src, dst, W, b, dinv):
    y = dinv[:, None] * (h @ W)
    yt = jnp.zeros((16, 4, NP), jnp.float32).at[:, :, :N].set(
        y.T.reshape(16, 4, N))
    out = _gcn_scatter(src, dst, yt)
    acct = out[:16] + out[16:]
    acc = acct.reshape(64, NP)[:, :N].T
    return dinv[:, None] * (acc + y) + b


@functools.partial(
    pl.kernel,
    out_type=(
        jax.ShapeDtypeStruct((ES8, HEADS, CPS8, CH), jnp.float32),
        jax.ShapeDtypeStruct((NW, NP), jnp.float32),
    ),
    mesh=_mesh(),
    compiler_params=_SC_PARAMS,
    scratch_types=[
        pltpu.VMEM((CH,), jnp.int32),
        pltpu.VMEM((CH,), jnp.int32),
        pltpu.VMEM((CH,), jnp.float32),
        pltpu.VMEM((1, NP), jnp.float32),
        pltpu.VMEM((1, NP), jnp.float32),
        pltpu.VMEM((1, NP), jnp.float32),
        pltpu.VMEM((1, NP), jnp.float32),
    ],
)
def _gat_edge(src_hbm, dst_hbm, asrc_hbm, adst_hbm, cbnd_hbm,
              ex_hbm, den_hbm, sidx, didx, exbuf, atab, btab, ctab, den):
    c = lax.axis_index("c")
    s = lax.axis_index("s")
    wid = s * NC + c
    h = wid // ES8
    es8 = wid % ES8
    pltpu.sync_copy(asrc_hbm.at[pl.ds(h, 1)], atab)
    pltpu.sync_copy(adst_hbm.at[pl.ds(h, 1)], btab)
    pltpu.sync_copy(cbnd_hbm.at[pl.ds(h, 1)], ctab)
    _zero_ref(den, 1, NP)
    zcol = jnp.zeros((16,), jnp.int32)

    def body(j, carry):
        base = es8 * EPS8 + j * CH
        pltpu.sync_copy(src_hbm.at[pl.ds(base, CH)], sidx)
        pltpu.sync_copy(dst_hbm.at[pl.ds(base, CH)], didx)
        for g in range(CH // 16):
            s16 = sidx[pl.ds(g * 16, 16)]
            d16 = didx[pl.ds(g * 16, 16)]
            av = plsc.load_gather(atab, [zcol, s16])
            bv = plsc.load_gather(btab, [zcol, d16])
            cv = plsc.load_gather(ctab, [zcol, d16])
            t = av + bv
            lr = jnp.maximum(t, 0.2 * t)
            exv = jnp.exp(lr - cv)
            exbuf[pl.ds(g * 16, 16)] = exv
            plsc.addupdate_scatter(den, [zcol, d16], exv)
        pltpu.sync_copy(exbuf, ex_hbm.at[es8, h, j])
        return carry

    lax.fori_loop(0, CPS8, body, 0)
    pltpu.sync_copy(den.at[0], den_hbm.at[wid])


def _make_gat_msg(k):
    @functools.partial(
        pl.kernel,
        out_type=jax.ShapeDtypeStruct((NW, 4, NP), jnp.float32),
        mesh=_mesh(),
        compiler_params=_SC_PARAMS,
        scratch_types=[
            pltpu.VMEM((CH,), jnp.int32),
            pltpu.VMEM((CH,), jnp.int32),
            pltpu.VMEM((CH,), jnp.float32),
            pltpu.VMEM((4, NP), jnp.float32),
            pltpu.VMEM((4, NP), jnp.float32),
        ],
    )
    def _gat_msg(src_hbm, dst_hbm, xgt_hbm, ex_hbm, out_hbm,
                 sidx, didx, exv, xtab, acc):
        c = lax.axis_index("c")
        s = lax.axis_index("s")
        wid = s * NC + c
        h = 2 * k + wid // 16
        cb = wid % 16
        pltpu.sync_copy(xgt_hbm.at[h, cb], xtab)
        _zero_ref(acc, 4, NP)
        cols = [jnp.full((16,), col, jnp.int32) for col in range(4)]

        def body(i, carry):
            es8 = i // CPS8
            j = i % CPS8
            base = i * CH
            pltpu.sync_copy(src_hbm.at[pl.ds(base, CH)], sidx)
            pltpu.sync_copy(dst_hbm.at[pl.ds(base, CH)], didx)
            pltpu.sync_copy(ex_hbm.at[es8, h, j], exv)
            for g in range(CH // 16):
                s16 = sidx[pl.ds(g * 16, 16)]
                d16 = didx[pl.ds(g * 16, 16)]
                w16 = exv[pl.ds(g * 16, 16)]
                for col in range(4):
                    v = plsc.load_gather(xtab, [cols[col], s16])
                    plsc.addupdate_scatter(acc, [cols[col], d16], v * w16)
            return carry

        lax.fori_loop(0, NCHUNK, body, 0)
        pltpu.sync_copy(acc, out_hbm.at[wid])

    return _gat_msg


_gat_msg0 = _make_gat_msg(0)
_gat_msg1 = _make_gat_msg(1)


def _gat_layer_sc(h, src, dst, Wg, att_src, att_dst, bg):
    xg = (h @ Wg).reshape(N, HEADS, HID)
    a_src = (xg * att_src[None, :, :]).sum(-1)
    a_dst = (xg * att_dst[None, :, :]).sum(-1)
    amax = a_src.max(axis=0)
    cbnd = jax.nn.leaky_relu(amax[None, :] + a_dst, 0.2)

    pad = lambda v: jnp.zeros((HEADS, NP), jnp.float32).at[:, :N].set(v.T)
    ex, den_part = _gat_edge(src, dst, pad(a_src), pad(a_dst), pad(cbnd))

    t_self = a_src + a_dst
    ex_self = jnp.exp(jax.nn.leaky_relu(t_self, 0.2) - cbnd)
    denom = den_part.reshape(HEADS, ES8, NP).sum(1)[:, :N].T + ex_self

    xgt = jnp.zeros((HEADS, 16, 4, NP), jnp.float32).at[:, :, :, :N].set(
        jnp.transpose(xg, (1, 2, 0)).reshape(HEADS, 16, 4, N))
    num0 = _gat_msg0(src, dst, xgt, ex)
    num1 = _gat_msg1(src, dst, xgt, ex)
    num = jnp.concatenate([
        num0.reshape(2, 16, 4, NP), num1.reshape(2, 16, 4, NP)], axis=0)
    num = num.reshape(HEADS, HID, NP)[:, :, :N]
    num = jnp.transpose(num, (2, 0, 1))
    num = num + ex_self[:, :, None] * xg
    out = (num / denom[:, :, None]).mean(axis=1) + bg
    return out


def _batch_norm(x, gamma, beta):
    mu = x.mean(axis=0)
    var = x.var(axis=0)
    return gamma * (x - mu) / jnp.sqrt(var + 1e-5) + beta


def _mlp_body(pooled_ref, fc1w_ref, fc1b_ref, fc2w_ref, fc2b_ref, out_ref):
    f = jnp.maximum(pooled_ref[...] @ fc1w_ref[...] + fc1b_ref[...], 0.0)
    out_ref[...] = f @ fc2w_ref[...] + fc2b_ref[...]


def kernel(x, edge_index, W1, b1, W2, b2, W3, b3, Wg, att_src, att_dst, bg,
           gamma, beta, fc1_w, fc1_b, fc2_w, fc2_b):
    src = edge_index[0]
    dst = edge_index[1]

    deg = _deg_kernel(dst).sum(0)[:N] + 1.0
    dinv = deg ** -0.5

    h = jax.nn.relu(_gcn_layer_sc(x, src, dst, W1, b1, dinv))
    h = _batch_norm(h, gamma, beta)
    h = jax.nn.relu(_gcn_layer_sc(h, src, dst, W2, b2, dinv))
    h = _batch_norm(h, gamma, beta)
    h = jax.nn.relu(_gcn_layer_sc(h, src, dst, W3, b3, dinv))
    h = jax.nn.relu(_gat_layer_sc(h, src, dst, Wg, att_src, att_dst, bg))
    pooled = jnp.mean(h, axis=0, keepdims=True)

    out = pl.pallas_call(
        _mlp_body,
        out_shape=jax.ShapeDtypeStruct((1, fc2_w.shape[1]), jnp.float32),
    )(pooled, fc1_w, fc1_b, fc2_w, fc2_b)
    return out


# speedup vs baseline: 12.6822x; 1.4749x over previous
import functools

import jax
import jax.numpy as jnp
from jax import lax
from jax.experimental import pallas as pl
from jax.experimental.pallas import tpu as pltpu
from jax.experimental.pallas import tpu_sc as plsc

N = 10000
E = 320000
HID = 64
HEADS = 4

NC = 2
NS = 16
NW = NC * NS
NP = 10240
CH = 2000
NCHUNK = E // CH
ES8 = 8
EPS8 = E // ES8
CPS8 = NCHUNK // ES8


def _mesh():
    return plsc.VectorSubcoreMesh(core_axis_name="c", subcore_axis_name="s")


_SC_PARAMS = pltpu.CompilerParams(needs_layout_passes=False)


def _zero_ref(ref, nrow, ncol):
    z = jnp.zeros((16,), jnp.float32)

    def zcol(col, carry):
        def zrow(i, carry2):
            ref[col, pl.ds(i * 16, 16)] = z
            return carry2
        lax.fori_loop(0, ncol // 16, zrow, 0)
        return carry
    lax.fori_loop(0, nrow, zcol, 0)


@functools.partial(
    pl.kernel,
    out_type=jax.ShapeDtypeStruct((NW, NP), jnp.float32),
    mesh=_mesh(),
    compiler_params=_SC_PARAMS,
    scratch_types=[
        pltpu.VMEM((CH,), jnp.int32),
        pltpu.VMEM((1, NP), jnp.float32),
    ],
)
def _deg_kernel(dst_hbm, out_hbm, didx, acc):
    c = lax.axis_index("c")
    s = lax.axis_index("s")
    wid = s * NC + c
    _zero_ref(acc, 1, NP)
    ones16 = jnp.ones((16,), jnp.float32)
    zcol = jnp.zeros((16,), jnp.int32)
    nch = NCHUNK // NW

    def body(i, carry):
        pltpu.sync_copy(dst_hbm.at[pl.ds((wid * nch + i) * CH, CH)], didx)
        for g in range(CH // 16):
            d16 = didx[pl.ds(g * 16, 16)]
            plsc.addupdate_scatter(acc, [zcol, d16], ones16)
        return carry

    lax.fori_loop(0, nch, body, 0)
    pltpu.sync_copy(acc.at[0], out_hbm.at[wid])


@functools.partial(
    pl.kernel,
    out_type=jax.ShapeDtypeStruct((NW, 4, NP), jnp.float32),
    mesh=_mesh(),
    compiler_params=_SC_PARAMS,
    scratch_types=[
        pltpu.VMEM((CH,), jnp.int32),
        pltpu.VMEM((CH,), jnp.int32),
        pltpu.VMEM((4, NP), jnp.float32),
        pltpu.VMEM((4, NP), jnp.float32),
    ],
)
def _gcn_scatter(src_hbm, dst_hbm, yt_hbm, out_hbm, sidx, didx, ytab, acc):
    c = lax.axis_index("c")
    s = lax.axis_index("s")
    wid = s * NC + c
    es2 = wid // 16
    cs = wid % 16
    pltpu.sync_copy(yt_hbm.at[cs], ytab)
    _zero_ref(acc, 4, NP)
    cols = [jnp.full((16,), col, jnp.int32) for col in range(4)]
    nch = NCHUNK // 2

    def body(i, carry):
        base = (es2 * nch + i) * CH
        pltpu.sync_copy(src_hbm.at[pl.ds(base, CH)], sidx)
        pltpu.sync_copy(dst_hbm.at[pl.ds(base, CH)], didx)
        for g in range(CH // 16):
            s16 = sidx[pl.ds(g * 16, 16)]
            d16 = didx[pl.ds(g * 16, 16)]
            for col in range(4):
                v = plsc.load_gather(ytab, [cols[col], s16])
                plsc.addupdate_scatter(acc, [cols[col], d16], v)
        return carry

    lax.fori_loop(0, nch, body, 0)
    pltpu.sync_copy(acc, out_hbm.at[wid])


def _gcn_layer_sc(h, src, dst, W, b, dinv):
    y = dinv[:, None] * (h @ W)
    yt = jnp.zeros((16, 4, NP), jnp.float32).at[:, :, :N].set(
        y.T.reshape(16, 4, N))
    out = _gcn_scatter(src, dst, yt)
    acct = out[:16] + out[16:]
    acc = acct.reshape(64, NP)[:, :N].T
    return dinv[:, None] * (acc + y) + b


@functools.partial(
    pl.kernel,
    out_type=(
        jax.ShapeDtypeStruct((ES8, HEADS, CPS8, CH), jnp.float32),
        jax.ShapeDtypeStruct((NW, NP), jnp.float32),
    ),
    mesh=_mesh(),
    compiler_params=_SC_PARAMS,
    scratch_types=[
        pltpu.VMEM((CH,), jnp.int32),
        pltpu.VMEM((CH,), jnp.int32),
        pltpu.VMEM((CH,), jnp.float32),
        pltpu.VMEM((1, NP), jnp.float32),
        pltpu.VMEM((1, NP), jnp.float32),
        pltpu.VMEM((1, NP), jnp.float32),
        pltpu.VMEM((1, NP), jnp.float32),
    ],
)
def _gat_edge(src_hbm, dst_hbm, asrc_hbm, adst_hbm, cbnd_hbm,
              ex_hbm, den_hbm, sidx, didx, exbuf, atab, btab, ctab, den):
    c = lax.axis_index("c")
    s = lax.axis_index("s")
    wid = s * NC + c
    h = wid // ES8
    es8 = wid % ES8
    pltpu.sync_copy(asrc_hbm.at[pl.ds(h, 1)], atab)
    pltpu.sync_copy(adst_hbm.at[pl.ds(h, 1)], btab)
    pltpu.sync_copy(cbnd_hbm.at[pl.ds(h, 1)], ctab)
    _zero_ref(den, 1, NP)
    zcol = jnp.zeros((16,), jnp.int32)

    def body(j, carry):
        base = es8 * EPS8 + j * CH
        pltpu.sync_copy(src_hbm.at[pl.ds(base, CH)], sidx)
        pltpu.sync_copy(dst_hbm.at[pl.ds(base, CH)], didx)
        for g in range(CH // 16):
            s16 = sidx[pl.ds(g * 16, 16)]
            d16 = didx[pl.ds(g * 16, 16)]
            av = plsc.load_gather(atab, [zcol, s16])
            bv = plsc.load_gather(btab, [zcol, d16])
            cv = plsc.load_gather(ctab, [zcol, d16])
            t = av + bv
            lr = jnp.maximum(t, 0.2 * t)
            exv = jnp.exp(lr - cv)
            exbuf[pl.ds(g * 16, 16)] = exv
            plsc.addupdate_scatter(den, [zcol, d16], exv)
        pltpu.sync_copy(exbuf, ex_hbm.at[es8, h, j])
        return carry

    lax.fori_loop(0, CPS8, body, 0)
    pltpu.sync_copy(den.at[0], den_hbm.at[wid])


def _make_gat_msg(k):
    @functools.partial(
        pl.kernel,
        out_type=jax.ShapeDtypeStruct((NW, 4, NP), jnp.float32),
        mesh=_mesh(),
        compiler_params=_SC_PARAMS,
        scratch_types=[
            pltpu.VMEM((CH,), jnp.int32),
            pltpu.VMEM((CH,), jnp.int32),
            pltpu.VMEM((CH,), jnp.float32),
            pltpu.VMEM((4, NP), jnp.float32),
            pltpu.VMEM((4, NP), jnp.float32),
        ],
    )
    def _gat_msg(src_hbm, dst_hbm, xgt_hbm, ex_hbm, out_hbm,
                 sidx, didx, exv, xtab, acc):
        c = lax.axis_index("c")
        s = lax.axis_index("s")
        wid = s * NC + c
        h = 2 * k + wid // 16
        cb = wid % 16
        pltpu.sync_copy(xgt_hbm.at[h, cb], xtab)
        _zero_ref(acc, 4, NP)
        cols = [jnp.full((16,), col, jnp.int32) for col in range(4)]

        def body(i, carry):
            es8 = i // CPS8
            j = i % CPS8
            base = i * CH
            pltpu.sync_copy(src_hbm.at[pl.ds(base, CH)], sidx)
            pltpu.sync_copy(dst_hbm.at[pl.ds(base, CH)], didx)
            pltpu.sync_copy(ex_hbm.at[es8, h, j], exv)
            for g in range(CH // 16):
                s16 = sidx[pl.ds(g * 16, 16)]
                d16 = didx[pl.ds(g * 16, 16)]
                w16 = exv[pl.ds(g * 16, 16)]
                for col in range(4):
                    v = plsc.load_gather(xtab, [cols[col], s16])
                    plsc.addupdate_scatter(acc, [cols[col], d16], v * w16)
            return carry

        lax.fori_loop(0, NCHUNK, body, 0)
        pltpu.sync_copy(acc, out_hbm.at[wid])

    return _gat_msg


_gat_msg0 = _make_gat_msg(0)
_gat_msg1 = _make_gat_msg(1)


def _gat_layer_sc(h, src, dst, Wg, att_src, att_dst, bg):
    xg = (h @ Wg).reshape(N, HEADS, HID)
    a_src = (xg * att_src[None, :, :]).sum(-1)
    a_dst = (xg * att_dst[None, :, :]).sum(-1)
    amax = a_src.max(axis=0)
    cbnd = jax.nn.leaky_relu(amax[None, :] + a_dst, 0.2)

    pad = lambda v: jnp.zeros((HEADS, NP), jnp.float32).at[:, :N].set(v.T)
    ex, den_part = _gat_edge(src, dst, pad(a_src), pad(a_dst), pad(cbnd))

    t_self = a_src + a_dst
    ex_self = jnp.exp(jax.nn.leaky_relu(t_self, 0.2) - cbnd)
    denom = den_part.reshape(HEADS, ES8, NP).sum(1)[:, :N].T + ex_self

    xgt = jnp.zeros((HEADS, 16, 4, NP), jnp.float32).at[:, :, :, :N].set(
        jnp.transpose(xg, (1, 2, 0)).reshape(HEADS, 16, 4, N))
    num0 = _gat_msg0(src, dst, xgt, ex)
    num1 = _gat_msg1(src, dst, xgt, ex)
    num = jnp.concatenate([
        num0.reshape(2, 16, 4, NP), num1.reshape(2, 16, 4, NP)], axis=0)
    num = num.reshape(HEADS, HID, NP)[:, :, :N]
    num = jnp.transpose(num, (2, 0, 1))
    num = num + ex_self[:, :, None] * xg
    out = (num / denom[:, :, None]).mean(axis=1) + bg
    return out


def _batch_norm(x, gamma, beta):
    mu = x.mean(axis=0)
    var = x.var(axis=0)
    return gamma * (x - mu) / jnp.sqrt(var + 1e-5) + beta


def _mlp_body(pooled_ref, fc1w_ref, fc1b_ref, fc2w_ref, fc2b_ref, out_ref):
    f = jnp.maximum(pooled_ref[...] @ fc1w_ref[...] + fc1b_ref[...], 0.0)
    out_ref[...] = f @ fc2w_ref[...] + fc2b_ref[...]


def kernel(x, edge_index, W1, b1, W2, b2, W3, b3, Wg, att_src, att_dst, bg,
           gamma, beta, fc1_w, fc1_b, fc2_w, fc2_b):
    src = edge_index[0]
    dst = edge_index[1]

    deg = _deg_kernel(dst).sum(0)[:N] + 1.0
    dinv = deg ** -0.5

    h = jax.nn.relu(_gcn_layer_sc(x, src, dst, W1, b1, dinv))
    h = _batch_norm(h, gamma, beta)
    h = jax.nn.relu(_gcn_layer_sc(h, src, dst, W2, b2, dinv))
    h = _batch_norm(h, gamma, beta)
    h = jax.nn.relu(_gcn_layer_sc(h, src, dst, W3, b3, dinv))
    h = jax.nn.relu(_gat_layer_sc(h, src, dst, Wg, att_src, att_dst, bg))
    pooled = jnp.mean(h, axis=0, keepdims=True)

    out = pl.pallas_call(
        _mlp_body,
        out_shape=jax.ShapeDtypeStruct((1, fc2_w.shape[1]), jnp.float32),
    )(pooled, fc1_w, fc1_b, fc2_w, fc2_b)
    return out


# speedup vs baseline: 12.7700x; 1.0069x over previous
import functools

import jax
import jax.numpy as jnp
from jax import lax
from jax.experimental import pallas as pl
from jax.experimental.pallas import tpu as pltpu
from jax.experimental.pallas import tpu_sc as plsc

N = 10000
E = 320000
HID = 64
HEADS = 4

NC = 2
NS = 16
NW = NC * NS
NP = 10240
CH = 2000
NCHUNK = E // CH
ES8 = 8
EPS8 = E // ES8
CPS8 = NCHUNK // ES8


def _mesh():
    return plsc.VectorSubcoreMesh(core_axis_name="c", subcore_axis_name="s")


_SC_PARAMS = pltpu.CompilerParams(needs_layout_passes=False)


def _zero_ref(ref, nrow, ncol):
    z = jnp.zeros((16,), jnp.float32)

    def zcol(col, carry):
        def zrow(i, carry2):
            ref[col, pl.ds(i * 16, 16)] = z
            return carry2
        lax.fori_loop(0, ncol // 16, zrow, 0)
        return carry
    lax.fori_loop(0, nrow, zcol, 0)


@functools.partial(
    pl.kernel,
    out_type=jax.ShapeDtypeStruct((NW, NP), jnp.float32),
    mesh=_mesh(),
    compiler_params=_SC_PARAMS,
    scratch_types=[
        pltpu.VMEM((CH,), jnp.int32),
        pltpu.VMEM((1, NP), jnp.float32),
    ],
)
def _deg_kernel(dst_hbm, out_hbm, didx, acc):
    c = lax.axis_index("c")
    s = lax.axis_index("s")
    wid = s * NC + c
    _zero_ref(acc, 1, NP)
    ones16 = jnp.ones((16,), jnp.float32)
    zcol = jnp.zeros((16,), jnp.int32)
    nch = NCHUNK // NW

    def body(i, carry):
        pltpu.sync_copy(dst_hbm.at[pl.ds((wid * nch + i) * CH, CH)], didx)
        for g in range(CH // 16):
            d16 = didx[pl.ds(g * 16, 16)]
            plsc.addupdate_scatter(acc, [zcol, d16], ones16)
        return carry

    lax.fori_loop(0, nch, body, 0)
    pltpu.sync_copy(acc.at[0], out_hbm.at[wid])


@functools.partial(
    pl.kernel,
    out_type=jax.ShapeDtypeStruct((NW, 4, NP), jnp.float32),
    mesh=_mesh(),
    compiler_params=_SC_PARAMS,
    scratch_types=[
        pltpu.VMEM((CH,), jnp.int32),
        pltpu.VMEM((CH,), jnp.int32),
        pltpu.VMEM((4, NP), jnp.float32),
        pltpu.VMEM((4, NP), jnp.float32),
    ],
)
def _gcn_scatter(src_hbm, dst_hbm, yt_hbm, out_hbm, sidx, didx, ytab, acc):
    c = lax.axis_index("c")
    s = lax.axis_index("s")
    wid = s * NC + c
    es2 = wid // 16
    cs = wid % 16
    pltpu.sync_copy(yt_hbm.at[cs], ytab)
    _zero_ref(acc, 4, NP)
    cols = [jnp.full((16,), col, jnp.int32) for col in range(4)]
    nch = NCHUNK // 2

    def body(i, carry):
        base = (es2 * nch + i) * CH
        pltpu.sync_copy(src_hbm.at[pl.ds(base, CH)], sidx)
        pltpu.sync_copy(dst_hbm.at[pl.ds(base, CH)], didx)
        for g in range(CH // 16):
            s16 = sidx[pl.ds(g * 16, 16)]
            d16 = didx[pl.ds(g * 16, 16)]
            for col in range(4):
                v = plsc.load_gather(ytab, [cols[col], s16])
                plsc.addupdate_scatter(acc, [cols[col], d16], v)
        return carry

    lax.fori_loop(0, nch, body, 0)
    pltpu.sync_copy(acc, out_hbm.at[wid])


def _pad_cols(v):
    return jnp.pad(v, ((0, 0), (0, NP - N)))


def _tc_pre_body(deg_ref, x_ref, w1_ref, yt_ref, dinv_ref):
    deg = jnp.sum(deg_ref[...], axis=0, keepdims=True) + 1.0
    dinv = lax.rsqrt(deg)
    dinv_ref[...] = dinv
    xt_w = lax.dot_general(w1_ref[...], x_ref[...],
                           (((0,), (1,)), ((), ())))
    yt_ref[...] = _pad_cols(xt_w) * dinv


def _tc_mid_body(acc_ref, yt_ref, dinv_ref, b_ref, gamma_ref, beta_ref,
                 w_ref, ynext_ref):
    acc = acc_ref[...].reshape(2, HID, NP)
    accT = acc[0] + acc[1]
    dinv = dinv_ref[...]
    h = jnp.maximum(dinv * (accT + yt_ref[...]) + b_ref[...].reshape(HID, 1),
                    0.0)
    hN = h[:, :N]
    mu = jnp.mean(hN, axis=1, keepdims=True)
    var = jnp.mean(hN * hN, axis=1, keepdims=True) - mu * mu
    hn = (gamma_ref[...].reshape(HID, 1) * (h - mu)
          / jnp.sqrt(var + 1e-5) + beta_ref[...].reshape(HID, 1))
    yn = lax.dot_general(w_ref[...], hn, (((0,), (0,)), ((), ())))
    ynext_ref[...] = yn * dinv


def _tc_gatprep_body(acc_ref, yt_ref, dinv_ref, b_ref, wg_ref,
                     asw_ref, adw_ref,
                     xgt_ref, asrc_ref, adst_ref, cbnd_ref, exself_ref):
    acc = acc_ref[...].reshape(2, HID, NP)
    accT = acc[0] + acc[1]
    h3 = jnp.maximum(dinv_ref[...] * (accT + yt_ref[...])
                     + b_ref[...].reshape(HID, 1), 0.0)
    xgt = lax.dot_general(wg_ref[...], h3, (((0,), (0,)), ((), ())))
    xgt_ref[...] = xgt
    for hh in range(HEADS):
        blk = xgt[hh * HID:(hh + 1) * HID, :]
        a_s = lax.dot_general(asw_ref[hh:hh + 1, :], blk,
                              (((1,), (0,)), ((), ())))
        a_d = lax.dot_general(adw_ref[hh:hh + 1, :], blk,
                              (((1,), (0,)), ((), ())))
        amax = jnp.max(a_s[:, :N], axis=1, keepdims=True)
        t = amax + a_d
        cb = jnp.maximum(t, 0.2 * t)
        ts = a_s + a_d
        lrs = jnp.maximum(ts, 0.2 * ts)
        asrc_ref[hh:hh + 1, :] = a_s
        adst_ref[hh:hh + 1, :] = a_d
        cbnd_ref[hh:hh + 1, :] = cb
        exself_ref[hh:hh + 1, :] = jnp.exp(lrs - cb)


def _tc_final_body(num0_ref, num1_ref, den_ref, exself_ref, xgt_ref,
                   bg_ref, fc1w_ref, fc1b_ref, fc2w_ref, fc2b_ref, out_ref):
    num0 = num0_ref[...].reshape(2, HID, NP)
    num1 = num1_ref[...].reshape(2, HID, NP)
    den = den_ref[...].reshape(HEADS, 8, NP).sum(axis=1)
    denom = den + exself_ref[...]
    gat = jnp.zeros((HID, NP), jnp.float32)
    for hh in range(HEADS):
        numh = num0[hh] if hh < 2 else num1[hh - 2]
        blk = xgt_ref[hh * HID:(hh + 1) * HID, :]
        gat = gat + (numh + exself_ref[hh:hh + 1, :] * blk) / denom[hh:hh + 1, :]
    gat = jnp.maximum(gat * (1.0 / HEADS) + bg_ref[...].reshape(HID, 1), 0.0)
    ones_row = jnp.ones((1, N), jnp.float32)
    pooled = lax.dot_general(ones_row, gat[:, :N],
                             (((1,), (1,)), ((), ()))) * (1.0 / N)
    f = jnp.maximum(pooled @ fc1w_ref[...] + fc1b_ref[...], 0.0)
    out_ref[...] = f @ fc2w_ref[...] + fc2b_ref[...]


def _tc_call(body, out_shapes, *args):
    return pl.pallas_call(
        body,
        out_shape=out_shapes,
    )(*args)


@functools.partial(
    pl.kernel,
    out_type=(
        jax.ShapeDtypeStruct((ES8, HEADS, CPS8, CH), jnp.float32),
        jax.ShapeDtypeStruct((NW, NP), jnp.float32),
    ),
    mesh=_mesh(),
    compiler_params=_SC_PARAMS,
    scratch_types=[
        pltpu.VMEM((CH,), jnp.int32),
        pltpu.VMEM((CH,), jnp.int32),
        pltpu.VMEM((CH,), jnp.float32),
        pltpu.VMEM((1, NP), jnp.float32),
        pltpu.VMEM((1, NP), jnp.float32),
        pltpu.VMEM((1, NP), jnp.float32),
        pltpu.VMEM((1, NP), jnp.float32),
    ],
)
def _gat_edge(src_hbm, dst_hbm, asrc_hbm, adst_hbm, cbnd_hbm,
              ex_hbm, den_hbm, sidx, didx, exbuf, atab, btab, ctab, den):
    c = lax.axis_index("c")
    s = lax.axis_index("s")
    wid = s * NC + c
    h = wid // ES8
    es8 = wid % ES8
    pltpu.sync_copy(asrc_hbm.at[pl.ds(h, 1)], atab)
    pltpu.sync_copy(adst_hbm.at[pl.ds(h, 1)], btab)
    pltpu.sync_copy(cbnd_hbm.at[pl.ds(h, 1)], ctab)
    _zero_ref(den, 1, NP)
    zcol = jnp.zeros((16,), jnp.int32)

    def body(j, carry):
        base = es8 * EPS8 + j * CH
        pltpu.sync_copy(src_hbm.at[pl.ds(base, CH)], sidx)
        pltpu.sync_copy(dst_hbm.at[pl.ds(base, CH)], didx)
        for g in range(CH // 16):
            s16 = sidx[pl.ds(g * 16, 16)]
            d16 = didx[pl.ds(g * 16, 16)]
            av = plsc.load_gather(atab, [zcol, s16])
            bv = plsc.load_gather(btab, [zcol, d16])
            cv = plsc.load_gather(ctab, [zcol, d16])
            t = av + bv
            lr = jnp.maximum(t, 0.2 * t)
            exv = jnp.exp(lr - cv)
            exbuf[pl.ds(g * 16, 16)] = exv
            plsc.addupdate_scatter(den, [zcol, d16], exv)
        pltpu.sync_copy(exbuf, ex_hbm.at[es8, h, j])
        return carry

    lax.fori_loop(0, CPS8, body, 0)
    pltpu.sync_copy(den.at[0], den_hbm.at[wid])


def _make_gat_msg(k):
    @functools.partial(
        pl.kernel,
        out_type=jax.ShapeDtypeStruct((NW, 4, NP), jnp.float32),
        mesh=_mesh(),
        compiler_params=_SC_PARAMS,
        scratch_types=[
            pltpu.VMEM((CH,), jnp.int32),
            pltpu.VMEM((CH,), jnp.int32),
            pltpu.VMEM((CH,), jnp.float32),
            pltpu.VMEM((4, NP), jnp.float32),
            pltpu.VMEM((4, NP), jnp.float32),
        ],
    )
    def _gat_msg(src_hbm, dst_hbm, xgt_hbm, ex_hbm, out_hbm,
                 sidx, didx, exv, xtab, acc):
        c = lax.axis_index("c")
        s = lax.axis_index("s")
        wid = s * NC + c
        h = 2 * k + wid // 16
        cb = wid % 16
        pltpu.sync_copy(xgt_hbm.at[h, cb], xtab)
        _zero_ref(acc, 4, NP)
        cols = [jnp.full((16,), col, jnp.int32) for col in range(4)]

        def body(i, carry):
            es8 = i // CPS8
            j = i % CPS8
            base = i * CH
            pltpu.sync_copy(src_hbm.at[pl.ds(base, CH)], sidx)
            pltpu.sync_copy(dst_hbm.at[pl.ds(base, CH)], didx)
            pltpu.sync_copy(ex_hbm.at[es8, h, j], exv)
            for g in range(CH // 16):
                s16 = sidx[pl.ds(g * 16, 16)]
                d16 = didx[pl.ds(g * 16, 16)]
                w16 = exv[pl.ds(g * 16, 16)]
                for col in range(4):
                    v = plsc.load_gather(xtab, [cols[col], s16])
                    plsc.addupdate_scatter(acc, [cols[col], d16], v * w16)
            return carry

        lax.fori_loop(0, NCHUNK, body, 0)
        pltpu.sync_copy(acc, out_hbm.at[wid])

    return _gat_msg


_gat_msg0 = _make_gat_msg(0)
_gat_msg1 = _make_gat_msg(1)


def _gat_layer_sc(src, dst, acc3, y3t, dinv, b3, Wg, att_src, att_dst, bg,
                  fc1_w, fc1_b, fc2_w, fc2_b):
    xgt, asrc, adst, cbnd, exself = pl.pallas_call(
        _tc_gatprep_body,
        out_shape=(
            jax.ShapeDtypeStruct((HEADS * HID, NP), jnp.float32),
            jax.ShapeDtypeStruct((HEADS, NP), jnp.float32),
            jax.ShapeDtypeStruct((HEADS, NP), jnp.float32),
            jax.ShapeDtypeStruct((HEADS, NP), jnp.float32),
            jax.ShapeDtypeStruct((HEADS, NP), jnp.float32),
        ),
    )(acc3, y3t, dinv, b3, Wg, att_src, att_dst)

    ex, den_part = _gat_edge(src, dst, asrc, adst, cbnd)
    xgt4 = xgt.reshape(HEADS, 16, 4, NP)
    num0 = _gat_msg0(src, dst, xgt4, ex)
    num1 = _gat_msg1(src, dst, xgt4, ex)

    out = pl.pallas_call(
        _tc_final_body,
        out_shape=jax.ShapeDtypeStruct((1, 6), jnp.float32),
    )(num0, num1, den_part, exself, xgt, bg, fc1_w, fc1_b, fc2_w, fc2_b)
    return out


def kernel(x, edge_index, W1, b1, W2, b2, W3, b3, Wg, att_src, att_dst, bg,
           gamma, beta, fc1_w, fc1_b, fc2_w, fc2_b):
    src = edge_index[0]
    dst = edge_index[1]

    deg_sc = _deg_kernel(dst)
    y1t, dinv = pl.pallas_call(
        _tc_pre_body,
        out_shape=(
            jax.ShapeDtypeStruct((HID, NP), jnp.float32),
            jax.ShapeDtypeStruct((1, NP), jnp.float32),
        ),
    )(deg_sc, x, W1)

    acc1 = _gcn_scatter(src, dst, y1t.reshape(16, 4, NP))
    y2t = pl.pallas_call(
        _tc_mid_body,
        out_shape=jax.ShapeDtypeStruct((HID, NP), jnp.float32),
    )(acc1, y1t, dinv, b1, gamma, beta, W2)

    acc2 = _gcn_scatter(src, dst, y2t.reshape(16, 4, NP))
    y3t = pl.pallas_call(
        _tc_mid_body,
        out_shape=jax.ShapeDtypeStruct((HID, NP), jnp.float32),
    )(acc2, y2t, dinv, b2, gamma, beta, W3)

    acc3 = _gcn_scatter(src, dst, y3t.reshape(16, 4, NP))
    return _gat_layer_sc(src, dst, acc3, y3t, dinv, b3, Wg, att_src, att_dst,
                         bg, fc1_w, fc1_b, fc2_w, fc2_b)
